# baseline (speedup 1.0000x reference)
import functools

import jax
import jax.numpy as jnp
from jax import lax
from jax.experimental import pallas as pl
from jax.experimental.pallas import tpu as pltpu
from jax.experimental.pallas import tpu_sc as plsc

DIM = 64
LANES = 16
CHUNK = 128


def kernel(u, i, j, user_factors, item_factors, item_biases):
    B = u.shape[0]
    info = plsc.get_sparse_core_info()
    nw = info.num_cores * info.num_subcores
    bpw = B // nw
    n_chunks = bpw // CHUNK

    uf2 = user_factors.reshape(-1, 2 * DIM)
    if2 = item_factors.reshape(-1, 2 * DIM)
    ib1 = item_biases.reshape(-1)

    mesh = plsc.VectorSubcoreMesh(core_axis_name="c", subcore_axis_name="s")

    @functools.partial(
        pl.kernel,
        mesh=mesh,
        out_type=jax.ShapeDtypeStruct((B,), jnp.float32),
        scratch_types=[
            pltpu.VMEM((bpw,), jnp.int32),
            pltpu.VMEM((bpw,), jnp.int32),
            pltpu.VMEM((bpw,), jnp.int32),
            pltpu.VMEM((bpw,), jnp.int32),
            pltpu.VMEM((bpw,), jnp.int32),
            pltpu.VMEM((bpw,), jnp.int32),
            pltpu.VMEM((CHUNK, 2 * DIM), jnp.float32),
            pltpu.VMEM((CHUNK, 2 * DIM), jnp.float32),
            pltpu.VMEM((CHUNK, 2 * DIM), jnp.float32),
            pltpu.VMEM((bpw,), jnp.float32),
            pltpu.VMEM((bpw,), jnp.float32),
            pltpu.VMEM((bpw,), jnp.float32),
            pltpu.SemaphoreType.DMA,
        ],
    )
    def sc_kernel(u_hbm, i_hbm, j_hbm, uf_hbm, if_hbm, ib_hbm, out_hbm,
                  u_idx, i_idx, j_idx, u_sh, i_sh, j_sh,
                  u_rows, i_rows, j_rows, bi_v, bj_v, out_v, sem):
        wid = lax.axis_index("s") * info.num_cores + lax.axis_index("c")
        base = wid * bpw

        pltpu.sync_copy(u_hbm.at[pl.ds(base, bpw)], u_idx)
        pltpu.sync_copy(i_hbm.at[pl.ds(base, bpw)], i_idx)
        pltpu.sync_copy(j_hbm.at[pl.ds(base, bpw)], j_idx)

        def shift_body(g, carry):
            sl = pl.ds(g * LANES, LANES)
            u_sh[sl] = lax.shift_right_logical(u_idx[sl], 1)
            i_sh[sl] = lax.shift_right_logical(i_idx[sl], 1)
            j_sh[sl] = lax.shift_right_logical(j_idx[sl], 1)
            return carry

        lax.fori_loop(0, bpw // LANES, shift_body, 0)

        bias_copies = []
        for c in range(n_chunks):
            sl = pl.ds(c * CHUNK, CHUNK)
            bias_copies.append(pltpu.async_copy(
                ib_hbm.at[i_idx.at[sl]], bi_v.at[sl], sem))
            bias_copies.append(pltpu.async_copy(
                ib_hbm.at[j_idx.at[sl]], bj_v.at[sl], sem))

        lane_iota = lax.iota(jnp.int32, LANES)
        perms = [jnp.bitwise_xor(lane_iota, jnp.full((LANES,), s, jnp.int32))
                 for s in (1, 2, 4, 8)]
        one16 = jnp.full((LANES,), 1, jnp.int32)

        def compute_chunk(c):
            def group_body(gg, carry):
                gb = c * CHUNK + gg * LANES
                pu = jnp.bitwise_and(u_idx[pl.ds(gb, LANES)], one16).astype(jnp.float32)
                pi = jnp.bitwise_and(i_idx[pl.ds(gb, LANES)], one16).astype(jnp.float32)
                pj = jnp.bitwise_and(j_idx[pl.ds(gb, LANES)], one16).astype(jnp.float32)
                acc = bi_v[pl.ds(gb, LANES)] - bj_v[pl.ds(gb, LANES)]
                for ee in range(LANES):
                    e = gg * LANES + ee
                    lane = jnp.full((LANES,), ee, jnp.int32)
                    fu = jnp.take(pu, lane)
                    fi = jnp.take(pi, lane)
                    fj = jnp.take(pj, lane)
                    p = None
                    for k in range(DIM // LANES):
                        lo = pl.ds(k * LANES, LANES)
                        hi = pl.ds(DIM + k * LANES, LANES)
                        ul = u_rows[e, lo]
                        il = i_rows[e, lo]
                        jl = j_rows[e, lo]
                        uv = ul + fu * (u_rows[e, hi] - ul)
                        iv = il + fi * (i_rows[e, hi] - il)
                        jv = jl + fj * (j_rows[e, hi] - jl)
                        t = uv * (iv - jv)
                        p = t if p is None else p + t
                    for perm in perms:
                        p = p + jnp.take(p, perm)
                    acc = jnp.where(lane_iota == ee, p + acc, acc)
                out_v[pl.ds(gb, LANES)] = acc
                return carry

            lax.fori_loop(0, CHUNK // LANES, group_body, 0)

        for cp in bias_copies:
            cp.wait()

        for c in range(n_chunks):
            sl = pl.ds(c * CHUNK, CHUNK)
            cp_u = pltpu.async_copy(uf_hbm.at[u_sh.at[sl]], u_rows, sem)
            cp_i = pltpu.async_copy(if_hbm.at[i_sh.at[sl]], i_rows, sem)
            cp_j = pltpu.async_copy(if_hbm.at[j_sh.at[sl]], j_rows, sem)
            cp_u.wait()
            cp_i.wait()
            cp_j.wait()
            compute_chunk(c)

        pltpu.sync_copy(out_v, out_hbm.at[pl.ds(base, bpw)])

    return sc_kernel(u, i, j, uf2, if2, ib1)


# speedup vs baseline: 1.5126x; 1.5126x over previous
import functools

import jax
import jax.numpy as jnp
from jax import lax
from jax.experimental import pallas as pl
from jax.experimental.pallas import tpu as pltpu
from jax.experimental.pallas import tpu_sc as plsc

DIM = 64
LANES = 16


def kernel(u, i, j, user_factors, item_factors, item_biases):
    B = u.shape[0]
    info = plsc.get_sparse_core_info()
    nw = info.num_cores * info.num_subcores
    bpw = B // nw

    ib1 = item_biases.reshape(-1)

    mesh = plsc.VectorSubcoreMesh(core_axis_name="c", subcore_axis_name="s")

    @functools.partial(
        pl.kernel,
        mesh=mesh,
        out_type=jax.ShapeDtypeStruct((B,), jnp.float32),
        scratch_types=[
            pltpu.VMEM((bpw,), jnp.int32),
            pltpu.VMEM((bpw,), jnp.int32),
            pltpu.VMEM((bpw,), jnp.int32),
            pltpu.SMEM((bpw,), jnp.int32),
            pltpu.SMEM((bpw,), jnp.int32),
            pltpu.SMEM((bpw,), jnp.int32),
            pltpu.VMEM((bpw // 2, DIM), jnp.float32),
            pltpu.VMEM((bpw // 2, DIM), jnp.float32),
            pltpu.VMEM((bpw // 2, DIM), jnp.float32),
            pltpu.VMEM((bpw,), jnp.float32),
            pltpu.VMEM((bpw,), jnp.float32),
            pltpu.VMEM((bpw,), jnp.float32),
            pltpu.SemaphoreType.DMA,
        ],
    )
    def sc_kernel(u_hbm, i_hbm, j_hbm, uf_hbm, if_hbm, ib_hbm, out_hbm,
                  u_idx, i_idx, j_idx, u_sb, i_sb, j_sb,
                  u_rows, i_rows, j_rows, bi_v, bj_v, out_v, sem):
        wid = lax.axis_index("s") * info.num_cores + lax.axis_index("c")
        base = wid * bpw

        pltpu.sync_copy(u_hbm.at[pl.ds(base, bpw)], u_idx)
        pltpu.sync_copy(i_hbm.at[pl.ds(base, bpw)], i_idx)
        pltpu.sync_copy(j_hbm.at[pl.ds(base, bpw)], j_idx)

        bias_copies = []
        for c in range(bpw // 128):
            sl = pl.ds(c * 128, 128)
            bias_copies.append(pltpu.async_copy(
                ib_hbm.at[i_idx.at[sl]], bi_v.at[sl], sem))
            bias_copies.append(pltpu.async_copy(
                ib_hbm.at[j_idx.at[sl]], bj_v.at[sl], sem))
        for cp in bias_copies:
            cp.wait()

        lane_iota = lax.iota(jnp.int32, LANES)
        perms = [jnp.bitwise_xor(lane_iota, jnp.full((LANES,), s, jnp.int32))
                 for s in (1, 2, 4, 8)]

        half = bpw // 2
        for h in range(2):
            hb = h * half

            def fetch_body(g, carry):
                gb = g * LANES
                uvec = u_idx[pl.ds(hb + gb, LANES)]
                ivec = i_idx[pl.ds(hb + gb, LANES)]
                jvec = j_idx[pl.ds(hb + gb, LANES)]
                for ee in range(LANES):
                    e = gb + ee
                    pltpu.async_copy(uf_hbm.at[uvec[ee]], u_rows.at[e], sem)
                    pltpu.async_copy(if_hbm.at[ivec[ee]], i_rows.at[e], sem)
                    pltpu.async_copy(if_hbm.at[jvec[ee]], j_rows.at[e], sem)
                return carry

            lax.fori_loop(0, half // LANES, fetch_body, 0)

            pltpu.make_async_copy(uf_hbm.at[pl.ds(0, half)], u_rows, sem).wait()
            pltpu.make_async_copy(if_hbm.at[pl.ds(0, half)], i_rows, sem).wait()
            pltpu.make_async_copy(if_hbm.at[pl.ds(0, half)], j_rows, sem).wait()

            def group_body(gg, carry):
                gb = gg * LANES
                acc = (bi_v[pl.ds(hb + gb, LANES)]
                       - bj_v[pl.ds(hb + gb, LANES)])
                for ee in range(LANES):
                    e = gb + ee
                    p = None
                    for k in range(DIM // LANES):
                        ksl = pl.ds(k * LANES, LANES)
                        t = u_rows[e, ksl] * (i_rows[e, ksl] - j_rows[e, ksl])
                        p = t if p is None else p + t
                    for perm in perms:
                        p = p + jnp.take(p, perm)
                    acc = jnp.where(lane_iota == ee, p + acc, acc)
                out_v[pl.ds(hb + gb, LANES)] = acc
                return carry

            lax.fori_loop(0, half // LANES, group_body, 0)

        pltpu.sync_copy(out_v, out_hbm.at[pl.ds(base, bpw)])

    return sc_kernel(u, i, j, user_factors, item_factors, ib1)
